# baseline (speedup 1.0000x reference)
import dataclasses
import functools

import jax
import jax.numpy as jnp
from jax import lax
from jax.experimental import pallas as pl
from jax.experimental.pallas import tpu as pltpu
from jax.experimental.pallas import tpu_sc as plsc

_SMEM11 = lambda: pl.BlockSpec((1, 1), lambda i: (0, 0),
                               memory_space=pltpu.MemorySpace.SMEM)

_N = 100000
_E = 3200000
_D = 128
_B = 64

_RB = 2048
_GA = 49
_NP = _RB * _GA
_PER_SC = _NP // 16
_RD = 2000
_GN = _N // _RD

_NW = 32
_C = 6400
_NIT = 16
_EW = _C * _NIT
_E2 = _EW * _NW


def _proj_body(x_ref, w_ref, b_ref, key_ref, val_ref, q_ref, kmax_ref):
    i = pl.program_id(0)
    y = jnp.dot(x_ref[...], w_ref[...], preferred_element_type=jnp.float32)
    y = y + b_ref[...]
    kcol = y[:, 0:1]
    key_ref[...] = kcol
    val_ref[...] = y[:, 1:2]
    q_ref[...] = y[:, 2:3]
    bm = jnp.max(jnp.abs(kcol))

    @pl.when(i == 0)
    def _():
        kmax_ref[0, 0] = bm

    @pl.when(i > 0)
    def _():
        kmax_ref[0, 0] = jnp.maximum(kmax_ref[0, 0], bm)


def _proj(x, wmat, brow):
    return pl.pallas_call(
        _proj_body,
        grid=(_GN,),
        in_specs=[
            pl.BlockSpec((_RD, _D), lambda i: (i, 0)),
            pl.BlockSpec((_D, _D), lambda i: (0, 0)),
            pl.BlockSpec((1, _D), lambda i: (0, 0)),
        ],
        out_specs=[
            pl.BlockSpec((_RD, 1), lambda i: (i, 0)),
            pl.BlockSpec((_RD, 1), lambda i: (i, 0)),
            pl.BlockSpec((_RD, 1), lambda i: (i, 0)),
            _SMEM11(),
        ],
        out_shape=[
            jax.ShapeDtypeStruct((_N, 1), jnp.float32),
            jax.ShapeDtypeStruct((_N, 1), jnp.float32),
            jax.ShapeDtypeStruct((_N, 1), jnp.float32),
            jax.ShapeDtypeStruct((1, 1), jnp.float32),
        ],
    )(x, wmat, brow)


def _sc_mesh():
    return plsc.VectorSubcoreMesh(core_axis_name="c", subcore_axis_name="s")


def _sc_params():
    cp = pltpu.CompilerParams()
    if "needs_layout_passes" in pltpu.CompilerParams.__dataclass_fields__:
        cp = dataclasses.replace(cp, needs_layout_passes=False)
    return cp


def _p1_body(src_hbm, dst_hbm, key_hbm, t16_hbm, z_hbm, out_hbm,
             srcb, dstb, kb, e1b, e2b, t16b, T1, T2):
    c = lax.axis_index("c")
    s = lax.axis_index("s")
    w = c * 16 + s
    pltpu.sync_copy(t16_hbm, t16b)
    rsl = pl.ds(s * _PER_SC, _PER_SC)
    pltpu.sync_copy(z_hbm, T1.at[rsl])
    pltpu.sync_copy(z_hbm, T2.at[rsl])
    plsc.subcore_barrier()
    tv = t16b[...]

    @pl.loop(0, _NIT)
    def _(it):
        base = pl.multiple_of(w * _EW + it * _C, 128)
        pltpu.sync_copy(src_hbm.at[pl.ds(base, _C)], srcb)
        pltpu.sync_copy(dst_hbm.at[pl.ds(base, _C)], dstb)
        pltpu.sync_copy(key_hbm.at[srcb], kb)

        @pl.loop(0, _C, step=16)
        def _(j):
            sl = pl.ds(j, 16)
            k16 = kb[sl] * tv
            e1b[sl] = jnp.exp(k16)
            e2b[sl] = jnp.exp(-k16)

        pltpu.sync_copy(e1b, T1.at[dstb], add=True)
        pltpu.sync_copy(e2b, T2.at[dstb], add=True)

    plsc.subcore_barrier()
    pltpu.sync_copy(T1.at[rsl], out_hbm.at[c, 0, rsl])
    pltpu.sync_copy(T2.at[rsl], out_hbm.at[c, 1, rsl])


def _p1(src, dst, key_flat, t16, zflat):
    kfn = pl.kernel(
        _p1_body,
        out_type=jax.ShapeDtypeStruct((2, 2, _NP), jnp.float32),
        mesh=_sc_mesh(),
        compiler_params=_sc_params(),
        scratch_types=[
            pltpu.VMEM((_C,), jnp.int32),
            pltpu.VMEM((_C,), jnp.int32),
            pltpu.VMEM((_C,), jnp.float32),
            pltpu.VMEM((_C,), jnp.float32),
            pltpu.VMEM((_C,), jnp.float32),
            pltpu.VMEM((16,), jnp.float32),
            pltpu.VMEM_SHARED((_NP,), jnp.float32),
            pltpu.VMEM_SHARED((_NP,), jnp.float32),
        ],
    )
    return kfn(src, dst, key_flat, t16, zflat)


def _stab_body(t1a_ref, t1b_ref, t2a_ref, t2b_ref, q_ref, invt_ref, m_ref):
    T1 = t1a_ref[...] + t1b_ref[...]
    T2 = t2a_ref[...] + t2b_ref[...]
    qv = q_ref[...]
    m = jnp.where(qv >= 0, qv * jnp.log(T1), -qv * jnp.log(T2))
    m = m * invt_ref[0, 0]
    m_ref[...] = jnp.where(T1 > 0, m, 0.0)


def _stab(t1a, t1b, t2a, t2b, qp, invt):
    blk = pl.BlockSpec((_RB, 1), lambda i: (i, 0))
    return pl.pallas_call(
        _stab_body,
        grid=(_GA,),
        in_specs=[blk, blk, blk, blk, blk, _SMEM11()],
        out_specs=pl.BlockSpec((_RB, 1), lambda i: (i, 0)),
        out_shape=jax.ShapeDtypeStruct((_NP, 1), jnp.float32),
    )(t1a, t1b, t2a, t2b, qp, invt)


def _p2_body(src_hbm, dst_hbm, key_hbm, val_hbm, q_hbm, m_hbm,
             z_hbm, out_hbm, srcb, dstb, kb, vb, qb, mb, e1b, e2b,
             S1, S2):
    c = lax.axis_index("c")
    s = lax.axis_index("s")
    w = c * 16 + s
    rsl = pl.ds(s * _PER_SC, _PER_SC)
    pltpu.sync_copy(z_hbm, S1.at[rsl])
    pltpu.sync_copy(z_hbm, S2.at[rsl])
    plsc.subcore_barrier()

    @pl.loop(0, _NIT)
    def _(it):
        base = pl.multiple_of(w * _EW + it * _C, 128)
        pltpu.sync_copy(src_hbm.at[pl.ds(base, _C)], srcb)
        pltpu.sync_copy(dst_hbm.at[pl.ds(base, _C)], dstb)
        pltpu.sync_copy(key_hbm.at[srcb], kb)
        pltpu.sync_copy(val_hbm.at[srcb], vb)
        pltpu.sync_copy(q_hbm.at[dstb], qb)
        pltpu.sync_copy(m_hbm.at[dstb], mb)

        @pl.loop(0, _C, step=16)
        def _(j):
            sl = pl.ds(j, 16)
            ex = jnp.exp(qb[sl] * kb[sl] - mb[sl])
            e1b[sl] = ex
            e2b[sl] = ex * vb[sl]

        pltpu.sync_copy(e1b, S1.at[dstb], add=True)
        pltpu.sync_copy(e2b, S2.at[dstb], add=True)

    plsc.subcore_barrier()
    pltpu.sync_copy(S1.at[rsl], out_hbm.at[c, 0, rsl])
    pltpu.sync_copy(S2.at[rsl], out_hbm.at[c, 1, rsl])


def _p2(src, dst, key_flat, val_flat, q_flat, m_flat, zflat):
    f32 = jnp.float32
    i32 = jnp.int32
    kfn = pl.kernel(
        _p2_body,
        out_type=jax.ShapeDtypeStruct((2, 2, _NP), jnp.float32),
        mesh=_sc_mesh(),
        compiler_params=_sc_params(),
        scratch_types=[
            pltpu.VMEM((_C,), i32), pltpu.VMEM((_C,), i32),
            pltpu.VMEM((_C,), f32), pltpu.VMEM((_C,), f32),
            pltpu.VMEM((_C,), f32), pltpu.VMEM((_C,), f32),
            pltpu.VMEM((_C,), f32), pltpu.VMEM((_C,), f32),
            pltpu.VMEM_SHARED((_NP,), f32),
            pltpu.VMEM_SHARED((_NP,), f32),
        ],
    )
    return kfn(src, dst, key_flat, val_flat, q_flat, m_flat, zflat)


def _score_body(s1a_ref, s1b_ref, s2a_ref, s2b_ref, aw_ref, ab_ref, sw_ref,
                cb_ref, score_ref, mmax_ref):
    i = pl.program_id(0)
    S1 = s1a_ref[...] + s1b_ref[...]
    S2 = s2a_ref[...] + s2b_ref[...]
    agg = S2 / (S1 + 1e-16)
    attn = jax.nn.gelu(agg) * aw_ref[0, 0] + ab_ref[0, 0]
    sc = attn * sw_ref[0, 0] + cb_ref[0, 0]
    rows = lax.broadcasted_iota(jnp.int32, (_RB, 1), 0) + i * _RB
    sc = jnp.where(rows < _N, sc, -1e30)
    score_ref[...] = sc
    bm = jnp.max(sc)

    @pl.when(i == 0)
    def _():
        mmax_ref[0, 0] = bm

    @pl.when(i > 0)
    def _():
        mmax_ref[0, 0] = jnp.maximum(mmax_ref[0, 0], bm)


def _score(s1a, s1b, s2a, s2b, aw, ab, sw, cb):
    blk = pl.BlockSpec((_RB, 1), lambda i: (i, 0))
    return pl.pallas_call(
        _score_body,
        grid=(_GA,),
        in_specs=[blk, blk, blk, blk, _SMEM11(), _SMEM11(), _SMEM11(),
                  _SMEM11()],
        out_specs=[pl.BlockSpec((_RB, 1), lambda i: (i, 0)),
                   _SMEM11()],
        out_shape=[jax.ShapeDtypeStruct((_NP, 1), jnp.float32),
                   jax.ShapeDtypeStruct((1, 1), jnp.float32)],
    )(s1a, s1b, s2a, s2b, aw, ab, sw, cb)


def _bsum_body(score_ref, batch_ref, mmax_ref, p_ref):
    i = pl.program_id(0)
    wv = jnp.exp(score_ref[...] - mmax_ref[0, 0])
    ids = lax.broadcasted_iota(jnp.int32, (_RB, 128), 1)
    oh = (batch_ref[...] == ids).astype(jnp.float32)
    contrib = lax.dot_general(wv, oh, (((0,), (0,)), ((), ())),
                              preferred_element_type=jnp.float32)

    @pl.when(i == 0)
    def _():
        p_ref[...] = contrib

    @pl.when(i > 0)
    def _():
        p_ref[...] = p_ref[...] + contrib


def _bsum(score, batchp, mmax):
    return pl.pallas_call(
        _bsum_body,
        grid=(_GA,),
        in_specs=[
            pl.BlockSpec((_RB, 1), lambda i: (i, 0)),
            pl.BlockSpec((_RB, 1), lambda i: (i, 0)),
            _SMEM11(),
        ],
        out_specs=pl.BlockSpec((1, 128), lambda i: (0, 0)),
        out_shape=jax.ShapeDtypeStruct((1, 128), jnp.float32),
    )(score, batchp, mmax)


def _pool_body(x_ref, score_ref, batch_ref, c_ref, num_ref, den_ref):
    i = pl.program_id(0)
    ids = lax.broadcasted_iota(jnp.int32, (_RD, 128), 1)
    oh = (batch_ref[...] == ids).astype(jnp.float32)
    crow = lax.dot_general(oh, c_ref[...], (((1,), (1,)), ((), ())),
                           preferred_element_type=jnp.float32)
    wv = jnp.exp(score_ref[...] - crow)
    ohw = oh * wv
    numc = lax.dot_general(ohw, x_ref[...], (((0,), (0,)), ((), ())),
                           preferred_element_type=jnp.float32)
    denc = jnp.sum(ohw, axis=0, keepdims=True)

    @pl.when(i == 0)
    def _():
        num_ref[...] = numc
        den_ref[...] = denc

    @pl.when(i > 0)
    def _():
        num_ref[...] = num_ref[...] + numc
        den_ref[...] = den_ref[...] + denc


def _pool(x, score, batch2d, c128):
    return pl.pallas_call(
        _pool_body,
        grid=(_GN,),
        in_specs=[
            pl.BlockSpec((_RD, _D), lambda i: (i, 0)),
            pl.BlockSpec((_RD, 1), lambda i: (i, 0)),
            pl.BlockSpec((_RD, 1), lambda i: (i, 0)),
            pl.BlockSpec((1, 128), lambda i: (0, 0)),
        ],
        out_specs=[pl.BlockSpec((128, 128), lambda i: (0, 0)),
                   pl.BlockSpec((1, 128), lambda i: (0, 0))],
        out_shape=[jax.ShapeDtypeStruct((128, 128), jnp.float32),
                   jax.ShapeDtypeStruct((1, 128), jnp.float32)],
    )(x, score, batch2d, c128)


def kernel(x, edge_index, batch, k_w, k_b, q_w, q_b, v_w, v_b,
           a_rel, m_rel, p_rel, a_w, a_b, w_pool, b_pool):
    f32 = jnp.float32
    c = a_rel[0, 0] * p_rel
    wmat = jnp.zeros((_D, _D), f32)
    wmat = wmat.at[:, 0].set(k_w[:, 0] * c)
    wmat = wmat.at[:, 1].set(v_w[:, 0] * m_rel[0, 0])
    wmat = wmat.at[:, 2].set(q_w[:, 0])
    brow = jnp.zeros((1, _D), f32)
    brow = brow.at[0, 0].set(k_b[0] * c)
    brow = brow.at[0, 1].set(v_b[0] * m_rel[0, 0])
    brow = brow.at[0, 2].set(q_b[0])

    key2, val2, q2, kmax = _proj(x, wmat, brow)

    t = 40.0 / jnp.maximum(kmax[0, 0], 1e-6)
    t16 = jnp.full((16,), t, f32)
    invt = (1.0 / t).reshape(1, 1)

    pad = _E2 - _E
    src = jnp.concatenate([edge_index[0], jnp.zeros((pad,), jnp.int32)])
    pad_dst = _N + (jnp.arange(pad, dtype=jnp.int32) % (_NP - _N))
    dst = jnp.concatenate([edge_index[1], pad_dst])

    key_flat = key2.reshape(_N)
    zflat = jnp.zeros((_PER_SC,), f32)

    T = _p1(src, dst, key_flat, t16, zflat)
    qp = jnp.pad(q2, ((0, _NP - _N), (0, 0)))
    m2 = _stab(T[0, 0].reshape(_NP, 1), T[1, 0].reshape(_NP, 1),
               T[0, 1].reshape(_NP, 1), T[1, 1].reshape(_NP, 1), qp, invt)

    S = _p2(src, dst, key_flat, val2.reshape(_N), q2.reshape(_N),
            m2.reshape(_NP), zflat)

    aw = a_w.reshape(1, 1)
    ab = a_b.reshape(1, 1)
    sw = jnp.sum(w_pool).reshape(1, 1)
    cb = (_D * b_pool[0]).reshape(1, 1)
    score, mmax = _score(S[0, 0].reshape(_NP, 1), S[1, 0].reshape(_NP, 1),
                         S[0, 1].reshape(_NP, 1), S[1, 1].reshape(_NP, 1),
                         aw, ab, sw, cb)

    batchp = jnp.pad(batch, (0, _NP - _N), constant_values=_B).reshape(_NP, 1)
    P = _bsum(score, batchp, mmax)

    pv = P[0, :_B]
    c64 = jnp.where(pv > 0, jnp.log(pv) + mmax[0, 0], 0.0)
    c128 = jnp.pad(c64, (0, 128 - _B)).reshape(1, 128)

    num, den = _pool(x, score, batch.reshape(_N, 1), c128)

    dcol = den[0, :_B]
    safe = jnp.where(dcol > 0, dcol, 1.0)
    pooled = num[:_B, :] / safe[:, None]
    return jnp.where(dcol[:, None] > 0, pooled, 0.0)


# speedup vs baseline: 1.5201x; 1.5201x over previous
import dataclasses
import functools

import jax
import jax.numpy as jnp
from jax import lax
from jax.experimental import pallas as pl
from jax.experimental.pallas import tpu as pltpu
from jax.experimental.pallas import tpu_sc as plsc

_SMEM11 = lambda: pl.BlockSpec((1, 1), lambda i: (0, 0),
                               memory_space=pltpu.MemorySpace.SMEM)

_N = 100000
_E = 3200000
_D = 128
_B = 64

_RB = 2048
_GA = 49
_NP = _RB * _GA
_PER_SC = _NP // 16
_RD = 2000
_GN = _N // _RD

_NW = 32
_C = 3584
_NIT = 28
_EW = _C * _NIT
_E2 = _EW * _NW


def _proj_body(x_ref, w_ref, b_ref, key_ref, val_ref, q_ref, kmax_ref):
    i = pl.program_id(0)
    y = jnp.dot(x_ref[...], w_ref[...], preferred_element_type=jnp.float32)
    y = y + b_ref[...]
    kcol = y[:, 0:1]
    key_ref[...] = kcol
    val_ref[...] = y[:, 1:2]
    q_ref[...] = y[:, 2:3]
    bm = jnp.max(jnp.abs(kcol))

    @pl.when(i == 0)
    def _():
        kmax_ref[0, 0] = bm

    @pl.when(i > 0)
    def _():
        kmax_ref[0, 0] = jnp.maximum(kmax_ref[0, 0], bm)


def _proj(x, wmat, brow):
    return pl.pallas_call(
        _proj_body,
        grid=(_GN,),
        in_specs=[
            pl.BlockSpec((_RD, _D), lambda i: (i, 0)),
            pl.BlockSpec((_D, _D), lambda i: (0, 0)),
            pl.BlockSpec((1, _D), lambda i: (0, 0)),
        ],
        out_specs=[
            pl.BlockSpec((_RD, 1), lambda i: (i, 0)),
            pl.BlockSpec((_RD, 1), lambda i: (i, 0)),
            pl.BlockSpec((_RD, 1), lambda i: (i, 0)),
            _SMEM11(),
        ],
        out_shape=[
            jax.ShapeDtypeStruct((_N, 1), jnp.float32),
            jax.ShapeDtypeStruct((_N, 1), jnp.float32),
            jax.ShapeDtypeStruct((_N, 1), jnp.float32),
            jax.ShapeDtypeStruct((1, 1), jnp.float32),
        ],
    )(x, wmat, brow)


def _sc_mesh():
    return plsc.VectorSubcoreMesh(core_axis_name="c", subcore_axis_name="s")


def _sc_params():
    cp = pltpu.CompilerParams()
    if "needs_layout_passes" in pltpu.CompilerParams.__dataclass_fields__:
        cp = dataclasses.replace(cp, needs_layout_passes=False)
    return cp


def _p1_body(src_hbm, dst_hbm, key_hbm, t16_hbm, z_hbm, out_hbm,
             srcb, dstb, kb, e1b, e2b, t16b, T1, T2):
    c = lax.axis_index("c")
    s = lax.axis_index("s")
    w = c * 16 + s
    pltpu.sync_copy(t16_hbm, t16b)
    rsl = pl.ds(s * _PER_SC, _PER_SC)
    pltpu.sync_copy(z_hbm, T1.at[rsl])
    pltpu.sync_copy(z_hbm, T2.at[rsl])
    plsc.subcore_barrier()
    tv = t16b[...]

    @pl.loop(0, _NIT)
    def _(it):
        base = pl.multiple_of(w * _EW + it * _C, 128)
        pltpu.sync_copy(src_hbm.at[pl.ds(base, _C)], srcb)
        pltpu.sync_copy(dst_hbm.at[pl.ds(base, _C)], dstb)
        pltpu.sync_copy(key_hbm.at[srcb], kb)

        @pl.loop(0, _C, step=16)
        def _(j):
            sl = pl.ds(j, 16)
            k16 = kb[sl] * tv
            e1b[sl] = jnp.exp(k16)
            e2b[sl] = jnp.exp(-k16)

        pltpu.sync_copy(e1b, T1.at[dstb], add=True)
        pltpu.sync_copy(e2b, T2.at[dstb], add=True)

    plsc.subcore_barrier()
    pltpu.sync_copy(T1.at[rsl], out_hbm.at[c, 0, rsl])
    pltpu.sync_copy(T2.at[rsl], out_hbm.at[c, 1, rsl])


def _p1(src, dst, key_flat, t16, zflat):
    kfn = pl.kernel(
        _p1_body,
        out_type=jax.ShapeDtypeStruct((2, 2, _NP), jnp.float32),
        mesh=_sc_mesh(),
        compiler_params=_sc_params(),
        scratch_types=[
            pltpu.VMEM((_C,), jnp.int32),
            pltpu.VMEM((_C,), jnp.int32),
            pltpu.VMEM((_C,), jnp.float32),
            pltpu.VMEM((_C,), jnp.float32),
            pltpu.VMEM((_C,), jnp.float32),
            pltpu.VMEM((16,), jnp.float32),
            pltpu.VMEM_SHARED((_NP,), jnp.float32),
            pltpu.VMEM_SHARED((_NP,), jnp.float32),
        ],
    )
    return kfn(src, dst, key_flat, t16, zflat)


def _stab_body(t1a_ref, t1b_ref, t2a_ref, t2b_ref, q_ref, invt_ref, m_ref):
    T1 = t1a_ref[...] + t1b_ref[...]
    T2 = t2a_ref[...] + t2b_ref[...]
    qv = q_ref[...]
    m = jnp.where(qv >= 0, qv * jnp.log(T1), -qv * jnp.log(T2))
    m = m * invt_ref[0, 0]
    m_ref[...] = jnp.where(T1 > 0, m, 0.0)


def _stab(t1a, t1b, t2a, t2b, qp, invt):
    blk = pl.BlockSpec((_RB, 1), lambda i: (i, 0))
    return pl.pallas_call(
        _stab_body,
        grid=(_GA,),
        in_specs=[blk, blk, blk, blk, blk, _SMEM11()],
        out_specs=pl.BlockSpec((_RB, 1), lambda i: (i, 0)),
        out_shape=jax.ShapeDtypeStruct((_NP, 1), jnp.float32),
    )(t1a, t1b, t2a, t2b, qp, invt)


def _p2_body(src_hbm, dst_hbm, key_hbm, val_hbm, q_hbm, m_hbm,
             z_hbm, out_hbm, srcb, dstb, kb, vb, qb, mb, e1b, e2b,
             S1, S2):
    c = lax.axis_index("c")
    s = lax.axis_index("s")
    w = c * 16 + s
    rsl = pl.ds(s * _PER_SC, _PER_SC)
    pltpu.sync_copy(z_hbm, S1.at[rsl])
    pltpu.sync_copy(z_hbm, S2.at[rsl])
    plsc.subcore_barrier()

    @pl.loop(0, _NIT)
    def _(it):
        base = pl.multiple_of(w * _EW + it * _C, 128)
        pltpu.sync_copy(src_hbm.at[pl.ds(base, _C)], srcb)
        pltpu.sync_copy(dst_hbm.at[pl.ds(base, _C)], dstb)
        pltpu.sync_copy(key_hbm.at[srcb], kb)
        pltpu.sync_copy(val_hbm.at[srcb], vb)
        pltpu.sync_copy(q_hbm.at[dstb], qb)
        pltpu.sync_copy(m_hbm.at[dstb], mb)

        @pl.loop(0, _C, step=16)
        def _(j):
            sl = pl.ds(j, 16)
            ex = jnp.exp(qb[sl] * kb[sl] - mb[sl])
            e1b[sl] = ex
            e2b[sl] = ex * vb[sl]

        pltpu.sync_copy(e1b, S1.at[dstb], add=True)
        pltpu.sync_copy(e2b, S2.at[dstb], add=True)

    plsc.subcore_barrier()
    pltpu.sync_copy(S1.at[rsl], out_hbm.at[c, 0, rsl])
    pltpu.sync_copy(S2.at[rsl], out_hbm.at[c, 1, rsl])


def _p2(src, dst, key_flat, val_flat, q_flat, m_flat, zflat):
    f32 = jnp.float32
    i32 = jnp.int32
    kfn = pl.kernel(
        _p2_body,
        out_type=jax.ShapeDtypeStruct((2, 2, _NP), jnp.float32),
        mesh=_sc_mesh(),
        compiler_params=_sc_params(),
        scratch_types=[
            pltpu.VMEM((_C,), i32), pltpu.VMEM((_C,), i32),
            pltpu.VMEM((_C,), f32), pltpu.VMEM((_C,), f32),
            pltpu.VMEM((_C,), f32), pltpu.VMEM((_C,), f32),
            pltpu.VMEM((_C,), f32), pltpu.VMEM((_C,), f32),
            pltpu.VMEM_SHARED((_NP,), f32),
            pltpu.VMEM_SHARED((_NP,), f32),
        ],
    )
    return kfn(src, dst, key_flat, val_flat, q_flat, m_flat, zflat)


def _score_body(s1a_ref, s1b_ref, s2a_ref, s2b_ref, aw_ref, ab_ref, sw_ref,
                cb_ref, score_ref, mmax_ref):
    i = pl.program_id(0)
    S1 = s1a_ref[...] + s1b_ref[...]
    S2 = s2a_ref[...] + s2b_ref[...]
    agg = S2 / (S1 + 1e-16)
    attn = jax.nn.gelu(agg) * aw_ref[0, 0] + ab_ref[0, 0]
    sc = attn * sw_ref[0, 0] + cb_ref[0, 0]
    rows = lax.broadcasted_iota(jnp.int32, (_RB, 1), 0) + i * _RB
    sc = jnp.where(rows < _N, sc, -1e30)
    score_ref[...] = sc
    bm = jnp.max(sc)

    @pl.when(i == 0)
    def _():
        mmax_ref[0, 0] = bm

    @pl.when(i > 0)
    def _():
        mmax_ref[0, 0] = jnp.maximum(mmax_ref[0, 0], bm)


def _score(s1a, s1b, s2a, s2b, aw, ab, sw, cb):
    blk = pl.BlockSpec((_RB, 1), lambda i: (i, 0))
    return pl.pallas_call(
        _score_body,
        grid=(_GA,),
        in_specs=[blk, blk, blk, blk, _SMEM11(), _SMEM11(), _SMEM11(),
                  _SMEM11()],
        out_specs=[pl.BlockSpec((_RB, 1), lambda i: (i, 0)),
                   _SMEM11()],
        out_shape=[jax.ShapeDtypeStruct((_NP, 1), jnp.float32),
                   jax.ShapeDtypeStruct((1, 1), jnp.float32)],
    )(s1a, s1b, s2a, s2b, aw, ab, sw, cb)


def _bsum_body(score_ref, batch_ref, mmax_ref, p_ref):
    i = pl.program_id(0)
    wv = jnp.exp(score_ref[...] - mmax_ref[0, 0])
    ids = lax.broadcasted_iota(jnp.int32, (_RB, 128), 1)
    oh = (batch_ref[...] == ids).astype(jnp.float32)
    contrib = lax.dot_general(wv, oh, (((0,), (0,)), ((), ())),
                              preferred_element_type=jnp.float32)

    @pl.when(i == 0)
    def _():
        p_ref[...] = contrib

    @pl.when(i > 0)
    def _():
        p_ref[...] = p_ref[...] + contrib


def _bsum(score, batchp, mmax):
    return pl.pallas_call(
        _bsum_body,
        grid=(_GA,),
        in_specs=[
            pl.BlockSpec((_RB, 1), lambda i: (i, 0)),
            pl.BlockSpec((_RB, 1), lambda i: (i, 0)),
            _SMEM11(),
        ],
        out_specs=pl.BlockSpec((1, 128), lambda i: (0, 0)),
        out_shape=jax.ShapeDtypeStruct((1, 128), jnp.float32),
    )(score, batchp, mmax)


def _pool_body(x_ref, score_ref, batch_ref, c_ref, num_ref, den_ref):
    i = pl.program_id(0)
    ids = lax.broadcasted_iota(jnp.int32, (_RD, 128), 1)
    oh = (batch_ref[...] == ids).astype(jnp.float32)
    crow = lax.dot_general(oh, c_ref[...], (((1,), (1,)), ((), ())),
                           preferred_element_type=jnp.float32)
    wv = jnp.exp(score_ref[...] - crow)
    ohw = oh * wv
    numc = lax.dot_general(ohw, x_ref[...], (((0,), (0,)), ((), ())),
                           preferred_element_type=jnp.float32)
    denc = jnp.sum(ohw, axis=0, keepdims=True)

    @pl.when(i == 0)
    def _():
        num_ref[...] = numc
        den_ref[...] = denc

    @pl.when(i > 0)
    def _():
        num_ref[...] = num_ref[...] + numc
        den_ref[...] = den_ref[...] + denc


def _pool(x, score, batch2d, c128):
    return pl.pallas_call(
        _pool_body,
        grid=(_GN,),
        in_specs=[
            pl.BlockSpec((_RD, _D), lambda i: (i, 0)),
            pl.BlockSpec((_RD, 1), lambda i: (i, 0)),
            pl.BlockSpec((_RD, 1), lambda i: (i, 0)),
            pl.BlockSpec((1, 128), lambda i: (0, 0)),
        ],
        out_specs=[pl.BlockSpec((128, 128), lambda i: (0, 0)),
                   pl.BlockSpec((1, 128), lambda i: (0, 0))],
        out_shape=[jax.ShapeDtypeStruct((128, 128), jnp.float32),
                   jax.ShapeDtypeStruct((1, 128), jnp.float32)],
    )(x, score, batch2d, c128)


def kernel(x, edge_index, batch, k_w, k_b, q_w, q_b, v_w, v_b,
           a_rel, m_rel, p_rel, a_w, a_b, w_pool, b_pool):
    f32 = jnp.float32
    c = a_rel[0, 0] * p_rel
    wmat = jnp.zeros((_D, _D), f32)
    wmat = wmat.at[:, 0].set(k_w[:, 0] * c)
    wmat = wmat.at[:, 1].set(v_w[:, 0] * m_rel[0, 0])
    wmat = wmat.at[:, 2].set(q_w[:, 0])
    brow = jnp.zeros((1, _D), f32)
    brow = brow.at[0, 0].set(k_b[0] * c)
    brow = brow.at[0, 1].set(v_b[0] * m_rel[0, 0])
    brow = brow.at[0, 2].set(q_b[0])

    key2, val2, q2, kmax = _proj(x, wmat, brow)

    t = 40.0 / jnp.maximum(kmax[0, 0], 1e-6)
    t16 = jnp.full((16,), t, f32)
    invt = (1.0 / t).reshape(1, 1)

    pad = _E2 - _E
    src = jnp.concatenate([edge_index[0], jnp.zeros((pad,), jnp.int32)])
    pad_dst = _N + (jnp.arange(pad, dtype=jnp.int32) % (_NP - _N))
    dst = jnp.concatenate([edge_index[1], pad_dst])

    key_flat = key2.reshape(_N)
    zflat = jnp.zeros((_PER_SC,), f32)

    T = _p1(src, dst, key_flat, t16, zflat)
    qp = jnp.pad(q2, ((0, _NP - _N), (0, 0)))
    m2 = _stab(T[0, 0].reshape(_NP, 1), T[1, 0].reshape(_NP, 1),
               T[0, 1].reshape(_NP, 1), T[1, 1].reshape(_NP, 1), qp, invt)

    S = _p2(src, dst, key_flat, val2.reshape(_N), q2.reshape(_N),
            m2.reshape(_NP), zflat)

    aw = a_w.reshape(1, 1)
    ab = a_b.reshape(1, 1)
    sw = jnp.sum(w_pool).reshape(1, 1)
    cb = (_D * b_pool[0]).reshape(1, 1)
    score, mmax = _score(S[0, 0].reshape(_NP, 1), S[1, 0].reshape(_NP, 1),
                         S[0, 1].reshape(_NP, 1), S[1, 1].reshape(_NP, 1),
                         aw, ab, sw, cb)

    batchp = jnp.pad(batch, (0, _NP - _N), constant_values=_B).reshape(_NP, 1)
    P = _bsum(score, batchp, mmax)

    pv = P[0, :_B]
    c64 = jnp.where(pv > 0, jnp.log(pv) + mmax[0, 0], 0.0)
    c128 = jnp.pad(c64, (0, 128 - _B)).reshape(1, 128)

    num, den = _pool(x, score, batch.reshape(_N, 1), c128)

    dcol = den[0, :_B]
    safe = jnp.where(dcol > 0, dcol, 1.0)
    pooled = num[:_B, :] / safe[:, None]
    return jnp.where(dcol[:, None] > 0, pooled, 0.0)


# speedup vs baseline: 1.5434x; 1.0153x over previous
import dataclasses
import functools

import jax
import jax.numpy as jnp
from jax import lax
from jax.experimental import pallas as pl
from jax.experimental.pallas import tpu as pltpu
from jax.experimental.pallas import tpu_sc as plsc

_SMEM11 = lambda: pl.BlockSpec((1, 1), lambda i: (0, 0),
                               memory_space=pltpu.MemorySpace.SMEM)

_N = 100000
_E = 3200000
_D = 128
_B = 64

_RB = 2048
_GA = 49
_NP = _RB * _GA
_PER_SC = _NP // 16
_RD = 2000
_GN = _N // _RD

_NW = 32
_C = 7168
_NIT = 14
_EW = _C * _NIT
_E2 = _EW * _NW


def _proj_body(x_ref, w_ref, b_ref, key_ref, val_ref, q_ref, kmax_ref):
    i = pl.program_id(0)
    y = jnp.dot(x_ref[...], w_ref[...], preferred_element_type=jnp.float32)
    y = y + b_ref[...]
    kcol = y[:, 0:1]
    key_ref[...] = kcol
    val_ref[...] = y[:, 1:2]
    q_ref[...] = y[:, 2:3]
    bm = jnp.max(jnp.abs(kcol))

    @pl.when(i == 0)
    def _():
        kmax_ref[0, 0] = bm

    @pl.when(i > 0)
    def _():
        kmax_ref[0, 0] = jnp.maximum(kmax_ref[0, 0], bm)


def _proj(x, wmat, brow):
    return pl.pallas_call(
        _proj_body,
        grid=(_GN,),
        in_specs=[
            pl.BlockSpec((_RD, _D), lambda i: (i, 0)),
            pl.BlockSpec((_D, _D), lambda i: (0, 0)),
            pl.BlockSpec((1, _D), lambda i: (0, 0)),
        ],
        out_specs=[
            pl.BlockSpec((_RD, 1), lambda i: (i, 0)),
            pl.BlockSpec((_RD, 1), lambda i: (i, 0)),
            pl.BlockSpec((_RD, 1), lambda i: (i, 0)),
            _SMEM11(),
        ],
        out_shape=[
            jax.ShapeDtypeStruct((_N, 1), jnp.float32),
            jax.ShapeDtypeStruct((_N, 1), jnp.float32),
            jax.ShapeDtypeStruct((_N, 1), jnp.float32),
            jax.ShapeDtypeStruct((1, 1), jnp.float32),
        ],
    )(x, wmat, brow)


def _sc_mesh():
    return plsc.VectorSubcoreMesh(core_axis_name="c", subcore_axis_name="s")


def _sc_params():
    cp = pltpu.CompilerParams()
    if "needs_layout_passes" in pltpu.CompilerParams.__dataclass_fields__:
        cp = dataclasses.replace(cp, needs_layout_passes=False)
    return cp


def _p1_body(src_hbm, dst_hbm, key_hbm, t16_hbm, z_hbm, out_hbm,
             srcb, dstb, kb, e1b, e2b, t16b, T1, T2):
    c = lax.axis_index("c")
    s = lax.axis_index("s")
    w = c * 16 + s
    pltpu.sync_copy(t16_hbm, t16b)
    rsl = pl.ds(s * _PER_SC, _PER_SC)
    pltpu.sync_copy(z_hbm, T1.at[rsl])
    pltpu.sync_copy(z_hbm, T2.at[rsl])
    plsc.subcore_barrier()
    tv = t16b[...]

    @pl.loop(0, _NIT)
    def _(it):
        base = pl.multiple_of(w * _EW + it * _C, 128)
        pltpu.sync_copy(src_hbm.at[pl.ds(base, _C)], srcb)
        pltpu.sync_copy(dst_hbm.at[pl.ds(base, _C)], dstb)
        pltpu.sync_copy(key_hbm.at[srcb], kb)

        @pl.loop(0, _C, step=16)
        def _(j):
            sl = pl.ds(j, 16)
            k16 = kb[sl] * tv
            e1b[sl] = jnp.exp(k16)
            e2b[sl] = jnp.exp(-k16)

        pltpu.sync_copy(e1b, T1.at[dstb], add=True)
        pltpu.sync_copy(e2b, T2.at[dstb], add=True)

    plsc.subcore_barrier()
    pltpu.sync_copy(T1.at[rsl], out_hbm.at[c, 0, rsl])
    pltpu.sync_copy(T2.at[rsl], out_hbm.at[c, 1, rsl])


def _p1(src, dst, key_flat, t16, zflat):
    kfn = pl.kernel(
        _p1_body,
        out_type=jax.ShapeDtypeStruct((2, 2, _NP), jnp.float32),
        mesh=_sc_mesh(),
        compiler_params=_sc_params(),
        scratch_types=[
            pltpu.VMEM((_C,), jnp.int32),
            pltpu.VMEM((_C,), jnp.int32),
            pltpu.VMEM((_C,), jnp.float32),
            pltpu.VMEM((_C,), jnp.float32),
            pltpu.VMEM((_C,), jnp.float32),
            pltpu.VMEM((16,), jnp.float32),
            pltpu.VMEM_SHARED((_NP,), jnp.float32),
            pltpu.VMEM_SHARED((_NP,), jnp.float32),
        ],
    )
    return kfn(src, dst, key_flat, t16, zflat)


def _stab_body(t1a_ref, t1b_ref, t2a_ref, t2b_ref, q_ref, invt_ref, m_ref):
    T1 = t1a_ref[...] + t1b_ref[...]
    T2 = t2a_ref[...] + t2b_ref[...]
    qv = q_ref[...]
    m = jnp.where(qv >= 0, qv * jnp.log(T1), -qv * jnp.log(T2))
    m = m * invt_ref[0, 0]
    m_ref[...] = jnp.where(T1 > 0, m, 0.0)


def _stab(t1a, t1b, t2a, t2b, qp, invt):
    blk = pl.BlockSpec((_RB, 1), lambda i: (i, 0))
    return pl.pallas_call(
        _stab_body,
        grid=(_GA,),
        in_specs=[blk, blk, blk, blk, blk, _SMEM11()],
        out_specs=pl.BlockSpec((_RB, 1), lambda i: (i, 0)),
        out_shape=jax.ShapeDtypeStruct((_NP, 1), jnp.float32),
    )(t1a, t1b, t2a, t2b, qp, invt)


def _p2_body(src_hbm, dst_hbm, key_hbm, val_hbm, q_hbm, m_hbm,
             z_hbm, out_hbm, srcb, dstb, kb, vb, qb, mb, e1b, e2b,
             S1, S2):
    c = lax.axis_index("c")
    s = lax.axis_index("s")
    w = c * 16 + s
    rsl = pl.ds(s * _PER_SC, _PER_SC)
    pltpu.sync_copy(z_hbm, S1.at[rsl])
    pltpu.sync_copy(z_hbm, S2.at[rsl])
    plsc.subcore_barrier()

    @pl.loop(0, _NIT)
    def _(it):
        base = pl.multiple_of(w * _EW + it * _C, 128)
        pltpu.sync_copy(src_hbm.at[pl.ds(base, _C)], srcb)
        pltpu.sync_copy(dst_hbm.at[pl.ds(base, _C)], dstb)
        pltpu.sync_copy(key_hbm.at[srcb], kb)
        pltpu.sync_copy(val_hbm.at[srcb], vb)
        pltpu.sync_copy(q_hbm.at[dstb], qb)
        pltpu.sync_copy(m_hbm.at[dstb], mb)

        @pl.loop(0, _C, step=16)
        def _(j):
            sl = pl.ds(j, 16)
            ex = jnp.exp(qb[sl] * kb[sl] - mb[sl])
            e1b[sl] = ex
            e2b[sl] = ex * vb[sl]

        pltpu.sync_copy(e1b, S1.at[dstb], add=True)
        pltpu.sync_copy(e2b, S2.at[dstb], add=True)

    plsc.subcore_barrier()
    pltpu.sync_copy(S1.at[rsl], out_hbm.at[c, 0, rsl])
    pltpu.sync_copy(S2.at[rsl], out_hbm.at[c, 1, rsl])


def _p2(src, dst, key_flat, val_flat, q_flat, m_flat, zflat):
    f32 = jnp.float32
    i32 = jnp.int32
    kfn = pl.kernel(
        _p2_body,
        out_type=jax.ShapeDtypeStruct((2, 2, _NP), jnp.float32),
        mesh=_sc_mesh(),
        compiler_params=_sc_params(),
        scratch_types=[
            pltpu.VMEM((_C,), i32), pltpu.VMEM((_C,), i32),
            pltpu.VMEM((_C,), f32), pltpu.VMEM((_C,), f32),
            pltpu.VMEM((_C,), f32), pltpu.VMEM((_C,), f32),
            pltpu.VMEM((_C,), f32), pltpu.VMEM((_C,), f32),
            pltpu.VMEM_SHARED((_NP,), f32),
            pltpu.VMEM_SHARED((_NP,), f32),
        ],
    )
    return kfn(src, dst, key_flat, val_flat, q_flat, m_flat, zflat)


def _score_body(s1a_ref, s1b_ref, s2a_ref, s2b_ref, aw_ref, ab_ref, sw_ref,
                cb_ref, score_ref, mmax_ref):
    i = pl.program_id(0)
    S1 = s1a_ref[...] + s1b_ref[...]
    S2 = s2a_ref[...] + s2b_ref[...]
    agg = S2 / (S1 + 1e-16)
    attn = jax.nn.gelu(agg) * aw_ref[0, 0] + ab_ref[0, 0]
    sc = attn * sw_ref[0, 0] + cb_ref[0, 0]
    rows = lax.broadcasted_iota(jnp.int32, (_RB, 1), 0) + i * _RB
    sc = jnp.where(rows < _N, sc, -1e30)
    score_ref[...] = sc
    bm = jnp.max(sc)

    @pl.when(i == 0)
    def _():
        mmax_ref[0, 0] = bm

    @pl.when(i > 0)
    def _():
        mmax_ref[0, 0] = jnp.maximum(mmax_ref[0, 0], bm)


def _score(s1a, s1b, s2a, s2b, aw, ab, sw, cb):
    blk = pl.BlockSpec((_RB, 1), lambda i: (i, 0))
    return pl.pallas_call(
        _score_body,
        grid=(_GA,),
        in_specs=[blk, blk, blk, blk, _SMEM11(), _SMEM11(), _SMEM11(),
                  _SMEM11()],
        out_specs=[pl.BlockSpec((_RB, 1), lambda i: (i, 0)),
                   _SMEM11()],
        out_shape=[jax.ShapeDtypeStruct((_NP, 1), jnp.float32),
                   jax.ShapeDtypeStruct((1, 1), jnp.float32)],
    )(s1a, s1b, s2a, s2b, aw, ab, sw, cb)


def _bsum_body(score_ref, batch_ref, mmax_ref, p_ref):
    i = pl.program_id(0)
    wv = jnp.exp(score_ref[...] - mmax_ref[0, 0])
    ids = lax.broadcasted_iota(jnp.int32, (_RB, 128), 1)
    oh = (batch_ref[...] == ids).astype(jnp.float32)
    contrib = lax.dot_general(wv, oh, (((0,), (0,)), ((), ())),
                              preferred_element_type=jnp.float32)

    @pl.when(i == 0)
    def _():
        p_ref[...] = contrib

    @pl.when(i > 0)
    def _():
        p_ref[...] = p_ref[...] + contrib


def _bsum(score, batchp, mmax):
    return pl.pallas_call(
        _bsum_body,
        grid=(_GA,),
        in_specs=[
            pl.BlockSpec((_RB, 1), lambda i: (i, 0)),
            pl.BlockSpec((_RB, 1), lambda i: (i, 0)),
            _SMEM11(),
        ],
        out_specs=pl.BlockSpec((1, 128), lambda i: (0, 0)),
        out_shape=jax.ShapeDtypeStruct((1, 128), jnp.float32),
    )(score, batchp, mmax)


def _pool_body(x_ref, score_ref, batch_ref, c_ref, num_ref, den_ref):
    i = pl.program_id(0)
    ids = lax.broadcasted_iota(jnp.int32, (_RD, 128), 1)
    oh = (batch_ref[...] == ids).astype(jnp.float32)
    crow = lax.dot_general(oh, c_ref[...], (((1,), (1,)), ((), ())),
                           preferred_element_type=jnp.float32)
    wv = jnp.exp(score_ref[...] - crow)
    ohw = oh * wv
    numc = lax.dot_general(ohw, x_ref[...], (((0,), (0,)), ((), ())),
                           preferred_element_type=jnp.float32)
    denc = jnp.sum(ohw, axis=0, keepdims=True)

    @pl.when(i == 0)
    def _():
        num_ref[...] = numc
        den_ref[...] = denc

    @pl.when(i > 0)
    def _():
        num_ref[...] = num_ref[...] + numc
        den_ref[...] = den_ref[...] + denc


def _pool(x, score, batch2d, c128):
    return pl.pallas_call(
        _pool_body,
        grid=(_GN,),
        in_specs=[
            pl.BlockSpec((_RD, _D), lambda i: (i, 0)),
            pl.BlockSpec((_RD, 1), lambda i: (i, 0)),
            pl.BlockSpec((_RD, 1), lambda i: (i, 0)),
            pl.BlockSpec((1, 128), lambda i: (0, 0)),
        ],
        out_specs=[pl.BlockSpec((128, 128), lambda i: (0, 0)),
                   pl.BlockSpec((1, 128), lambda i: (0, 0))],
        out_shape=[jax.ShapeDtypeStruct((128, 128), jnp.float32),
                   jax.ShapeDtypeStruct((1, 128), jnp.float32)],
    )(x, score, batch2d, c128)


def kernel(x, edge_index, batch, k_w, k_b, q_w, q_b, v_w, v_b,
           a_rel, m_rel, p_rel, a_w, a_b, w_pool, b_pool):
    f32 = jnp.float32
    c = a_rel[0, 0] * p_rel
    wmat = jnp.zeros((_D, _D), f32)
    wmat = wmat.at[:, 0].set(k_w[:, 0] * c)
    wmat = wmat.at[:, 1].set(v_w[:, 0] * m_rel[0, 0])
    wmat = wmat.at[:, 2].set(q_w[:, 0])
    brow = jnp.zeros((1, _D), f32)
    brow = brow.at[0, 0].set(k_b[0] * c)
    brow = brow.at[0, 1].set(v_b[0] * m_rel[0, 0])
    brow = brow.at[0, 2].set(q_b[0])

    key2, val2, q2, kmax = _proj(x, wmat, brow)

    t = 40.0 / jnp.maximum(kmax[0, 0], 1e-6)
    t16 = jnp.full((16,), t, f32)
    invt = (1.0 / t).reshape(1, 1)

    pad = _E2 - _E
    src = jnp.concatenate([edge_index[0], jnp.zeros((pad,), jnp.int32)])
    pad_dst = _N + (jnp.arange(pad, dtype=jnp.int32) % (_NP - _N))
    dst = jnp.concatenate([edge_index[1], pad_dst])

    key_flat = key2.reshape(_N)
    zflat = jnp.zeros((_PER_SC,), f32)

    T = _p1(src, dst, key_flat, t16, zflat)
    qp = jnp.pad(q2, ((0, _NP - _N), (0, 0)))
    m2 = _stab(T[0, 0].reshape(_NP, 1), T[1, 0].reshape(_NP, 1),
               T[0, 1].reshape(_NP, 1), T[1, 1].reshape(_NP, 1), qp, invt)

    S = _p2(src, dst, key_flat, val2.reshape(_N), q2.reshape(_N),
            m2.reshape(_NP), zflat)

    aw = a_w.reshape(1, 1)
    ab = a_b.reshape(1, 1)
    sw = jnp.sum(w_pool).reshape(1, 1)
    cb = (_D * b_pool[0]).reshape(1, 1)
    score, mmax = _score(S[0, 0].reshape(_NP, 1), S[1, 0].reshape(_NP, 1),
                         S[0, 1].reshape(_NP, 1), S[1, 1].reshape(_NP, 1),
                         aw, ab, sw, cb)

    batchp = jnp.pad(batch, (0, _NP - _N), constant_values=_B).reshape(_NP, 1)
    P = _bsum(score, batchp, mmax)

    pv = P[0, :_B]
    c64 = jnp.where(pv > 0, jnp.log(pv) + mmax[0, 0], 0.0)
    c128 = jnp.pad(c64, (0, 128 - _B)).reshape(1, 128)

    num, den = _pool(x, score, batch.reshape(_N, 1), c128)

    dcol = den[0, :_B]
    safe = jnp.where(dcol > 0, dcol, 1.0)
    pooled = num[:_B, :] / safe[:, None]
    return jnp.where(dcol[:, None] > 0, pooled, 0.0)
